# baseline (speedup 1.0000x reference)
import functools

import jax
import jax.numpy as jnp
from jax import lax
from jax.experimental import pallas as pl
from jax.experimental.pallas import tpu as pltpu
from jax.experimental.pallas import tpu_sc as plsc

EMB = 64
MAX_SEQ = 200
BATCH = 4096
N_FLAT = BATCH * MAX_SEQ
NC, NS = 2, 16
NW = NC * NS
SPAN = N_FLAT // NW
CHUNK = 256
N_CHUNKS = SPAN // CHUNK
IDXW = 128
N_STREAMS = CHUNK // IDXW
IDX_ROWS_PER_WORKER = SPAN // IDXW


def _combo_body(l_ref, p_ref, o_ref):
    pos = p_ref[...]
    o_ref[...] = jnp.concatenate(
        [pos + l_ref[l, :][None, :] for l in range(4)], axis=0)


def _make_combo(label_table, pos_table):
    return pl.pallas_call(
        _combo_body,
        out_shape=jax.ShapeDtypeStruct((4 * MAX_SEQ, EMB), jnp.float32),
    )(label_table, pos_table)


@functools.partial(
    pl.kernel,
    out_type=jax.ShapeDtypeStruct((N_FLAT, 2 * EMB), jnp.float32),
    mesh=plsc.VectorSubcoreMesh(core_axis_name="c", subcore_axis_name="s"),
    scratch_types=[
        pltpu.VMEM((N_STREAMS, IDXW), jnp.int32),
        pltpu.VMEM((N_STREAMS, IDXW), jnp.int32),
        pltpu.VMEM((N_STREAMS, IDXW), jnp.int32),
        pltpu.VMEM((N_STREAMS, IDXW), jnp.int32),
        pltpu.VMEM((CHUNK, EMB), jnp.float32),
        pltpu.VMEM((CHUNK, EMB), jnp.float32),
        pltpu.VMEM((CHUNK, EMB), jnp.float32),
        pltpu.VMEM((CHUNK, EMB), jnp.float32),
        pltpu.SemaphoreType.DMA,
        pltpu.SemaphoreType.DMA,
        pltpu.SemaphoreType.DMA,
        pltpu.SemaphoreType.DMA,
        pltpu.SemaphoreType.DMA,
        pltpu.SemaphoreType.DMA,
    ],
    compiler_params=pltpu.CompilerParams(use_tc_tiling_on_sc=False),
)
def _sc_lookup(ttab, combo, tidx, labl, out,
               ti0, ti1, ci0, ci1, a0, a1, b0, b1,
               si0, si1, sg0, sg1, so0, so1):
    wid = lax.axis_index("s") * NC + lax.axis_index("c")
    row0 = wid * IDX_ROWS_PER_WORKER
    base0 = wid * SPAN
    ti, ci, abuf, bbuf = (ti0, ti1), (ci0, ci1), (a0, a1), (b0, b1)
    si, sg, so = (si0, si1), (sg0, sg1), (so0, so1)

    def idx_rows(k):
        return pl.ds(row0 + k * N_STREAMS, N_STREAMS)

    def issue_idx(k, s):
        pltpu.async_copy(tidx.at[idx_rows(k)], ti[s], si[s])
        pltpu.async_copy(labl.at[idx_rows(k)], ci[s], si[s])

    def wait_idx(k, s):
        pltpu.make_async_copy(tidx.at[idx_rows(k)], ti[s], si[s]).wait()
        pltpu.make_async_copy(labl.at[idx_rows(k)], ci[s], si[s]).wait()

    def compute_cidx(k, s):
        base = base0 + k * CHUNK
        for j in range(N_STREAMS):
            for q in range(IDXW // 16):
                sl = pl.ds(q * 16, 16)
                off = base + j * IDXW + q * 16
                posv = (lax.iota(jnp.int32, 16) + off) % MAX_SEQ
                ci[s][j, sl] = ci[s][j, sl] * MAX_SEQ + posv

    def issue_gathers(k, s):
        for j in range(N_STREAMS):
            d = pl.ds(j * IDXW, IDXW)
            pltpu.async_copy(ttab.at[ti[s].at[j]], abuf[s].at[d], sg[s])
            pltpu.async_copy(combo.at[ci[s].at[j]], bbuf[s].at[d], sg[s])

    def wait_gathers(k, s):
        for j in range(N_STREAMS):
            d = pl.ds(j * IDXW, IDXW)
            pltpu.make_async_copy(ttab.at[ti[s].at[j]], abuf[s].at[d], sg[s]).wait()
            pltpu.make_async_copy(combo.at[ci[s].at[j]], bbuf[s].at[d], sg[s]).wait()

    def out_dst(k):
        return out.at[pl.ds(base0 + k * CHUNK, CHUNK), pl.ds(0, EMB)]

    def issue_out(k, s):
        pltpu.async_copy(abuf[s], out_dst(k), so[s])

    def wait_out(k, s):
        pltpu.make_async_copy(abuf[s], out_dst(k), so[s]).wait()

    def add_loop(s):
        a, b = abuf[s], bbuf[s]

        def row_body(r, _):
            for q in range(EMB // 16):
                sl = pl.ds(q * 16, 16)
                a[r, sl] = (a[r, sl] + b[r, sl]) * (1.0 / 3.0)
            return _

        lax.fori_loop(0, CHUNK, row_body, 0)

    issue_idx(0, 0)
    issue_idx(1, 1)
    wait_idx(0, 0)
    compute_cidx(0, 0)
    issue_gathers(0, 0)

    def steady(k, par):
        nxt = 1 - par
        wait_idx(k + 1, nxt)
        compute_cidx(k + 1, nxt)
        wait_gathers(k, par)
        issue_idx(k + 2, par)

        @pl.when(k >= 1)
        def _():
            wait_out(k - 1, nxt)

        issue_gathers(k + 1, nxt)
        add_loop(par)
        issue_out(k, par)

    def fori_body(c2, carry):
        for par in (0, 1):
            steady(2 * c2 + par, par)
        return carry

    lax.fori_loop(0, (N_CHUNKS - 2) // 2, fori_body, 0)

    k = N_CHUNKS - 2
    wait_gathers(k, 0)
    wait_idx(k + 1, 1)
    compute_cidx(k + 1, 1)
    wait_out(k - 1, 1)
    issue_gathers(k + 1, 1)
    add_loop(0)
    issue_out(k, 0)
    k = N_CHUNKS - 1
    wait_gathers(k, 1)
    add_loop(1)
    issue_out(k, 1)
    wait_out(k - 1, 0)
    wait_out(k, 1)


def kernel(label, time_idx, time_table, label_table, pos_table):
    tidx = time_idx.astype(jnp.int32).reshape(N_FLAT // IDXW, IDXW)
    labl = label.astype(jnp.int32).reshape(N_FLAT // IDXW, IDXW)
    combo = _make_combo(label_table, pos_table)
    out = _sc_lookup(time_table, combo, tidx, labl)
    return out[:, :EMB].reshape(BATCH, MAX_SEQ, EMB)


# speedup vs baseline: 1.0036x; 1.0036x over previous
import functools

import jax
import jax.numpy as jnp
from jax import lax
from jax.experimental import pallas as pl
from jax.experimental.pallas import tpu as pltpu
from jax.experimental.pallas import tpu_sc as plsc

EMB = 64
MAX_SEQ = 200
BATCH = 4096
N_FLAT = BATCH * MAX_SEQ
NC, NS = 2, 16
NW = NC * NS
SPAN = N_FLAT // NW
CHUNK = 256
N_CHUNKS = SPAN // CHUNK
IDXW = 128
N_STREAMS = CHUNK // IDXW
IDX_ROWS_PER_WORKER = SPAN // IDXW


def _combo_body(l_ref, p_ref, o_ref):
    pos = p_ref[...]
    o_ref[...] = jnp.concatenate(
        [pos + l_ref[l, :][None, :] for l in range(4)], axis=0)


def _make_combo(label_table, pos_table):
    return pl.pallas_call(
        _combo_body,
        out_shape=jax.ShapeDtypeStruct((4 * MAX_SEQ, EMB), jnp.float32),
    )(label_table, pos_table)


@functools.partial(
    pl.kernel,
    out_type=jax.ShapeDtypeStruct((N_FLAT, 2 * EMB), jnp.float32),
    mesh=plsc.VectorSubcoreMesh(core_axis_name="c", subcore_axis_name="s"),
    scratch_types=[
        pltpu.VMEM((N_STREAMS, IDXW), jnp.int32),
        pltpu.VMEM((N_STREAMS, IDXW), jnp.int32),
        pltpu.VMEM((N_STREAMS, IDXW), jnp.int32),
        pltpu.VMEM((N_STREAMS, IDXW), jnp.int32),
        pltpu.VMEM((CHUNK, EMB), jnp.float32),
        pltpu.VMEM((CHUNK, EMB), jnp.float32),
        pltpu.VMEM((CHUNK, EMB), jnp.float32),
        pltpu.VMEM((CHUNK, EMB), jnp.float32),
        pltpu.SemaphoreType.DMA,
        pltpu.SemaphoreType.DMA,
        pltpu.SemaphoreType.DMA,
        pltpu.SemaphoreType.DMA,
        pltpu.SemaphoreType.DMA,
        pltpu.SemaphoreType.DMA,
    ],
    compiler_params=pltpu.CompilerParams(use_tc_tiling_on_sc=False),
)
def _sc_lookup(ttab, combo, tidx, labl, out,
               ti0, ti1, ci0, ci1, a0, a1, b0, b1,
               si0, si1, sg0, sg1, so0, so1):
    wid = lax.axis_index("s") * NC + lax.axis_index("c")
    row0 = wid * IDX_ROWS_PER_WORKER
    base0 = wid * SPAN
    ti, ci, abuf, bbuf = (ti0, ti1), (ci0, ci1), (a0, a1), (b0, b1)
    si, sg, so = (si0, si1), (sg0, sg1), (so0, so1)

    def idx_rows(k):
        return pl.ds(row0 + k * N_STREAMS, N_STREAMS)

    def issue_idx(k, s):
        pltpu.async_copy(tidx.at[idx_rows(k)], ti[s], si[s])
        pltpu.async_copy(labl.at[idx_rows(k)], ci[s], si[s])

    def wait_idx(k, s):
        pltpu.make_async_copy(tidx.at[idx_rows(k)], ti[s], si[s]).wait()
        pltpu.make_async_copy(labl.at[idx_rows(k)], ci[s], si[s]).wait()

    def compute_cidx(k, s):
        base = base0 + k * CHUNK
        for j in range(N_STREAMS):
            for q in range(IDXW // 16):
                sl = pl.ds(q * 16, 16)
                off = base + j * IDXW + q * 16
                posv = (lax.iota(jnp.int32, 16) + off) % MAX_SEQ
                ci[s][j, sl] = ci[s][j, sl] * MAX_SEQ + posv

    def issue_gathers(k, s):
        for j in range(N_STREAMS):
            d = pl.ds(j * IDXW, IDXW)
            pltpu.async_copy(ttab.at[ti[s].at[j]], abuf[s].at[d], sg[s])
            pltpu.async_copy(combo.at[ci[s].at[j]], bbuf[s].at[d], sg[s])

    def wait_gathers(k, s):
        for j in range(N_STREAMS):
            d = pl.ds(j * IDXW, IDXW)
            pltpu.make_async_copy(ttab.at[ti[s].at[j]], abuf[s].at[d], sg[s]).wait()
            pltpu.make_async_copy(combo.at[ci[s].at[j]], bbuf[s].at[d], sg[s]).wait()

    def out_dst(k):
        return out.at[pl.ds(base0 + k * CHUNK, CHUNK), pl.ds(0, EMB)]

    def issue_out(k, s):
        pltpu.async_copy(abuf[s], out_dst(k), so[s])

    def wait_out(k, s):
        pltpu.make_async_copy(abuf[s], out_dst(k), so[s]).wait()

    def add_loop(s):
        a, b = abuf[s], bbuf[s]

        def row_body(r, _):
            for q in range(EMB // 16):
                sl = pl.ds(q * 16, 16)
                a[r, sl] = (a[r, sl] + b[r, sl]) * (1.0 / 3.0)
            return _

        lax.fori_loop(0, CHUNK, row_body, 0)

    issue_idx(0, 0)
    issue_idx(1, 1)
    wait_idx(0, 0)
    compute_cidx(0, 0)
    issue_gathers(0, 0)

    def steady(k, par):
        nxt = 1 - par
        wait_idx(k + 1, nxt)
        compute_cidx(k + 1, nxt)
        wait_gathers(k, par)
        issue_idx(k + 2, par)

        @pl.when(k >= 1)
        def _():
            wait_out(k - 1, nxt)

        issue_gathers(k + 1, nxt)
        add_loop(par)
        issue_out(k, par)

    def fori_body(c2, carry):
        for par in (0, 1):
            steady(2 * c2 + par, par)
        return carry

    lax.fori_loop(0, (N_CHUNKS - 2) // 2, fori_body, 0)

    k = N_CHUNKS - 2
    wait_gathers(k, 0)
    wait_idx(k + 1, 1)
    compute_cidx(k + 1, 1)
    wait_out(k - 1, 1)
    issue_gathers(k + 1, 1)
    add_loop(0)
    issue_out(k, 0)
    k = N_CHUNKS - 1
    wait_gathers(k, 1)
    add_loop(1)
    issue_out(k, 1)
    wait_out(k - 1, 0)
    wait_out(k, 1)


def kernel(label, time_idx, time_table, label_table, pos_table):
    tidx = time_idx.astype(jnp.int32).reshape(N_FLAT // IDXW, IDXW)
    labl32 = lax.optimization_barrier(label.astype(jnp.int32))
    labl = labl32.reshape(N_FLAT // IDXW, IDXW)
    combo = _make_combo(label_table, pos_table)
    out = _sc_lookup(time_table, combo, tidx, labl)
    return out[:, :EMB].reshape(BATCH, MAX_SEQ, EMB)
